# baseline (speedup 1.0000x reference)
import functools

import jax
import jax.numpy as jnp
from jax import lax
from jax.experimental import pallas as pl
from jax.experimental.pallas import tpu as pltpu
from jax.experimental.pallas import tpu_sc as plsc

DIM = 4096
NUM_EXPERTS = 64
EPSILON = 1e-06
BLOCK_N = 512
N_TOKENS = 8192
NUM_SC = 2
NUM_SUBCORES = 16
DENOM_CHUNK = N_TOKENS // NUM_SUBCORES
OUT_CHUNK = N_TOKENS // (NUM_SC * NUM_SUBCORES)


def _router_block(x_ref, w_ref, b_ref, p_ref, idx_ref):
    logits = jax.lax.dot_general(
        x_ref[:], w_ref[:], (((1,), (1,)), ((), ())),
        preferred_element_type=jnp.float32) + b_ref[:]
    m = jnp.max(logits, axis=1, keepdims=True)
    e = jnp.exp(logits - m)
    s = jnp.sum(e, axis=1)
    idx = jnp.argmax(logits, axis=1)
    p_ref[:] = (1.0 / s).reshape(1, 1, BLOCK_N)
    idx_ref[:] = idx.astype(jnp.int32).reshape(1, 1, BLOCK_N)


def _router(xf, W, b2):
    nblk = N_TOKENS // BLOCK_N
    return pl.pallas_call(
        _router_block,
        grid=(nblk,),
        in_specs=[
            pl.BlockSpec((BLOCK_N, DIM), lambda i: (i, 0)),
            pl.BlockSpec((NUM_EXPERTS, DIM), lambda i: (0, 0)),
            pl.BlockSpec((1, NUM_EXPERTS), lambda i: (0, 0)),
        ],
        out_specs=[
            pl.BlockSpec((1, 1, BLOCK_N), lambda i: (i, 0, 0)),
            pl.BlockSpec((1, 1, BLOCK_N), lambda i: (i, 0, 0)),
        ],
        out_shape=[
            jax.ShapeDtypeStruct((nblk, 1, BLOCK_N), jnp.float32),
            jax.ShapeDtypeStruct((nblk, 1, BLOCK_N), jnp.int32),
        ],
    )(xf, W, b2)


def _sc_gate(p_hbm, idx_hbm, out_hbm, pv, iv, denom_v, zbuf, rows, sh_denom):
    s_idx = lax.axis_index("s")
    c_idx = lax.axis_index("c")
    zeros16 = jnp.zeros((16,), jnp.float32)

    for t in range(4):
        base = s_idx * DENOM_CHUNK + t * 128
        pltpu.sync_copy(p_hbm.at[pl.ds(base, 128)], pv.at[t])
        pltpu.sync_copy(idx_hbm.at[pl.ds(base, 128)], iv.at[t])

    @pl.when(s_idx == 0)
    def _zero_shared():
        for k in range(4):
            zbuf[pl.ds(k * 16, 16)] = zeros16
        pltpu.sync_copy(zbuf, sh_denom)

    plsc.subcore_barrier()

    for t in range(4):
        pltpu.sync_copy(pv.at[t], sh_denom.at[iv.at[t]], add=True)

    plsc.subcore_barrier()
    pltpu.sync_copy(sh_denom, denom_v)

    def _zb(i, _):
        for k in range(16):
            rows[pl.ds(i * 256 + k * 16, 16)] = zeros16
        return 0

    lax.fori_loop(0, 64, _zb, 0)

    for r2 in range(2):
        r = 2 * c_idx + r2
        for j in range(8):
            ivec = iv[r, pl.ds(j * 16, 16)]
            pvec = pv[r, pl.ds(j * 16, 16)]
            dg = plsc.load_gather(denom_v, [ivec])
            val = pvec * (float(N_TOKENS) / (dg + EPSILON))
            tok = r2 * 128 + j * 16 + lax.broadcasted_iota(jnp.int32, (16,), 0)
            pos = tok * NUM_EXPERTS + ivec
            plsc.store_scatter(rows, [pos], val)

    out_base = (s_idx * DENOM_CHUNK + c_idx * OUT_CHUNK) * NUM_EXPERTS
    pltpu.sync_copy(rows, out_hbm.at[pl.ds(out_base, OUT_CHUNK * NUM_EXPERTS)])


_sc_gate_call = functools.partial(
    pl.kernel,
    mesh=plsc.VectorSubcoreMesh(core_axis_name="c", subcore_axis_name="s"),
    out_type=jax.ShapeDtypeStruct((N_TOKENS * NUM_EXPERTS,), jnp.float32),
    scratch_types=[
        pltpu.VMEM((4, 128), jnp.float32),
        pltpu.VMEM((4, 128), jnp.int32),
        pltpu.VMEM((NUM_EXPERTS,), jnp.float32),
        pltpu.VMEM((NUM_EXPERTS,), jnp.float32),
        pltpu.VMEM((OUT_CHUNK * NUM_EXPERTS,), jnp.float32),
        pltpu.VMEM_SHARED((NUM_EXPERTS,), jnp.float32),
    ],
    compiler_params=pltpu.CompilerParams(needs_layout_passes=False),
)(_sc_gate)


def kernel(x, W, b):
    batch_size, seq_len, dim = x.shape
    n = batch_size * seq_len
    xf = x.reshape(n, dim)
    b2 = b.reshape(1, NUM_EXPERTS)
    p3, idx3 = _router(xf, W, b2)
    out = _sc_gate_call(p3.reshape(n), idx3.reshape(n))
    return out.reshape(batch_size, seq_len, NUM_EXPERTS)
